# baseline (speedup 1.0000x reference)
import functools

import jax
import jax.numpy as jnp
from jax.experimental import pallas as pl
from jax.experimental.pallas import tpu as pltpu
from jax.experimental.pallas import tpu_sc as plsc

N = 10000
E = 320000
D = 64
DG = 16
NC = 2
NS = 16
NT = NC * NS
CK = 128
CH = 80
EPT = CH * CK
EPAD = NT * EPT
VPAD = 10112
STRIPE = VPAD // NS
DUMMY_DST = 10008


def _mesh():
    return plsc.VectorSubcoreMesh(core_axis_name="c", subcore_axis_name="s")


_SC_PARAMS = pltpu.CompilerParams(use_tc_tiling_on_sc=False)


def _sc_degree(dst2, ones, z16):
    @functools.partial(
        pl.kernel,
        out_type=jax.ShapeDtypeStruct((NC, VPAD, DG), jnp.float32),
        mesh=_mesh(),
        scratch_types=[
            pltpu.VMEM((CH, CK), jnp.int32),
            pltpu.VMEM((CK, DG), jnp.float32),
            pltpu.VMEM_SHARED((VPAD, DG), jnp.float32),
            pltpu.SemaphoreType.DMA((4,)),
        ],
        compiler_params=_SC_PARAMS,
    )
    def k(dst_hbm, ones_hbm, z_hbm, out_hbm, didx_all, ones_v, acc, ssem):
        c = jax.lax.axis_index("c")
        s = jax.lax.axis_index("s")
        w = c * NS + s
        pltpu.sync_copy(ones_hbm, ones_v)
        pltpu.sync_copy(dst_hbm.at[pl.ds(w * CH, CH)], didx_all)
        pltpu.sync_copy(z_hbm.at[pl.ds(s * STRIPE, STRIPE)],
                        acc.at[pl.ds(s * STRIPE, STRIPE)])
        plsc.subcore_barrier()

        for b in range(4):
            pltpu.async_copy(ones_v, acc.at[didx_all.at[b]], ssem.at[b],
                             add=True)

        @pl.loop(0, CH // 4 - 1)
        def _(t):
            base = t * 4
            for b in range(4):
                pltpu.make_async_copy(ones_v, acc.at[didx_all.at[base + b]],
                                      ssem.at[b]).wait()
                pltpu.async_copy(ones_v, acc.at[didx_all.at[base + 4 + b]],
                                 ssem.at[b], add=True)

        for b in range(4):
            pltpu.make_async_copy(ones_v, acc.at[didx_all.at[b]],
                                  ssem.at[b]).wait()

        plsc.subcore_barrier()
        pltpu.sync_copy(acc.at[pl.ds(s * STRIPE, STRIPE)],
                        out_hbm.at[c].at[pl.ds(s * STRIPE, STRIPE)])

    return k(dst2, ones, z16)


NB = 2


def _sc_agg(src2, dst2, g, z64):
    @functools.partial(
        pl.kernel,
        out_type=jax.ShapeDtypeStruct((NC, VPAD, D), jnp.float32),
        mesh=_mesh(),
        scratch_types=[
            pltpu.VMEM((CH, CK), jnp.int32),
            pltpu.VMEM((CH, CK), jnp.int32),
            pltpu.VMEM((NB, CK, D), jnp.float32),
            pltpu.VMEM_SHARED((VPAD, D), jnp.float32),
            pltpu.VMEM_SHARED((VPAD, D), jnp.float32),
            pltpu.SemaphoreType.DMA((NB,)),
            pltpu.SemaphoreType.DMA((NB,)),
            pltpu.SemaphoreType.DMA((4,)),
        ],
        compiler_params=_SC_PARAMS,
    )
    def k(src_hbm, dst_hbm, g_hbm, z_hbm, out_hbm, sidx_all, didx_all,
          rows_v, g_sp, acc, gsem, ssem, psem):
        c = jax.lax.axis_index("c")
        s = jax.lax.axis_index("s")
        w = c * NS + s
        pltpu.async_copy(src_hbm.at[pl.ds(w * CH, CH)], sidx_all, psem.at[0])
        pltpu.async_copy(dst_hbm.at[pl.ds(w * CH, CH)], didx_all, psem.at[1])
        pltpu.async_copy(g_hbm.at[pl.ds(s * STRIPE, STRIPE)],
                         g_sp.at[pl.ds(s * STRIPE, STRIPE)], psem.at[2])
        pltpu.async_copy(z_hbm.at[pl.ds(s * STRIPE, STRIPE)],
                         acc.at[pl.ds(s * STRIPE, STRIPE)], psem.at[3])
        pltpu.make_async_copy(src_hbm.at[pl.ds(w * CH, CH)], sidx_all,
                              psem.at[0]).wait()
        pltpu.make_async_copy(dst_hbm.at[pl.ds(w * CH, CH)], didx_all,
                              psem.at[1]).wait()
        pltpu.make_async_copy(g_hbm.at[pl.ds(s * STRIPE, STRIPE)],
                              g_sp.at[pl.ds(s * STRIPE, STRIPE)],
                              psem.at[2]).wait()
        pltpu.make_async_copy(z_hbm.at[pl.ds(s * STRIPE, STRIPE)],
                              acc.at[pl.ds(s * STRIPE, STRIPE)],
                              psem.at[3]).wait()
        plsc.subcore_barrier()

        def gsrc(i_mod4):
            return g_hbm if i_mod4 == 0 else g_sp

        def fire_g(base, off, b):
            pltpu.async_copy(gsrc(off % 4).at[sidx_all.at[base + off]],
                             rows_v.at[b], gsem.at[b])

        def wait_g(base, off, b):
            pltpu.make_async_copy(gsrc(off % 4).at[sidx_all.at[base + off]],
                                  rows_v.at[b], gsem.at[b]).wait()

        def fire_s(base, off, b):
            pltpu.async_copy(rows_v.at[b], acc.at[didx_all.at[base + off]],
                             ssem.at[b], add=True)

        def wait_s(base, off, b):
            pltpu.make_async_copy(rows_v.at[b],
                                  acc.at[didx_all.at[base + off]],
                                  ssem.at[b]).wait()

        for b in range(NB):
            fire_g(0, b, b)

        @pl.loop(0, CH // (2 * NB) - 1)
        def _(t):
            base4 = t * 2 * NB
            for half in range(2):
                base = half * NB
                for b in range(NB):
                    wait_g(base4, base + b, b)
                    fire_s(base4, base + b, b)
                for b in range(NB):
                    wait_s(base4, base + b, b)
                    fire_g(base4, base + NB + b, b)

        base4 = CH - 2 * NB
        for half in range(2):
            base = half * NB
            for b in range(NB):
                wait_g(base4, base + b, b)
                fire_s(base4, base + b, b)
            for b in range(NB):
                wait_s(base4, base + b, b)
                if half == 0:
                    fire_g(base4, base + NB + b, b)

        plsc.subcore_barrier()
        pltpu.sync_copy(acc.at[pl.ds(s * STRIPE, STRIPE)],
                        out_hbm.at[c].at[pl.ds(s * STRIPE, STRIPE)])

    return k(src2, dst2, g, z64)


def _tc_mm(x, W):
    def body(x_ref, w_ref, o_ref):
        o_ref[...] = jnp.dot(x_ref[...], w_ref[...],
                             preferred_element_type=jnp.float32)

    return pl.pallas_call(
        body,
        out_shape=jax.ShapeDtypeStruct((x.shape[0], W.shape[1]), jnp.float32),
    )(x, W)


def _tc_scale0(degp, h0):
    def body(dp_ref, h_ref, dinv_ref, g_ref):
        dp = dp_ref[...]
        deg = dp[0, :N, 0:1] + dp[1, :N, 0:1] + 1.0
        dinv = jax.lax.rsqrt(deg)
        dinv_ref[...] = dinv
        g_ref[...] = jnp.concatenate(
            [h_ref[...] * dinv, jnp.zeros((VPAD - N, D), jnp.float32)])

    return pl.pallas_call(
        body,
        out_shape=[
            jax.ShapeDtypeStruct((N, 1), jnp.float32),
            jax.ShapeDtypeStruct((VPAD, D), jnp.float32),
        ],
    )(degp, h0)


def _tc_mid(p, g, dinv, b, W):
    def body(p_ref, g_ref, dinv_ref, b_ref, w_ref, o_ref):
        pv = p_ref[...]
        sm = pv[0, :N] + pv[1, :N] + g_ref[...][:N]
        dv = dinv_ref[...]
        f = jnp.maximum(dv * sm + b_ref[...], 0.0)
        o_ref[...] = jnp.concatenate(
            [dv * jnp.dot(f, w_ref[...], preferred_element_type=jnp.float32),
             jnp.zeros((VPAD - N, D), jnp.float32)])

    return pl.pallas_call(
        body,
        out_shape=jax.ShapeDtypeStruct((VPAD, D), jnp.float32),
    )(p, g, dinv, b, W)


def _tc_last(p, g, dinv, b):
    def body(p_ref, g_ref, dinv_ref, b_ref, o_ref):
        pv = p_ref[...]
        sm = pv[0, :N] + pv[1, :N] + g_ref[...][:N]
        o_ref[...] = dinv_ref[...] * sm + b_ref[...]

    return pl.pallas_call(
        body,
        out_shape=jax.ShapeDtypeStruct((N, D), jnp.float32),
    )(p, g, dinv, b)


def kernel(x, edge_index, W0, b0, W1, b1, W2, b2, W3, b3):
    src = edge_index[0].astype(jnp.int32)
    dst = edge_index[1].astype(jnp.int32)
    pad = EPAD - E
    src2 = jnp.concatenate([src, jnp.zeros((pad,), jnp.int32)]).reshape(
        NT * CH, CK)
    dst2 = jnp.concatenate([dst, jnp.full((pad,), DUMMY_DST, jnp.int32)]
                           ).reshape(NT * CH, CK)
    ones = jnp.ones((CK, DG), jnp.float32)
    z16 = jnp.zeros((VPAD, DG), jnp.float32)
    z64 = jnp.zeros((VPAD, D), jnp.float32)

    degp = _sc_degree(dst2, ones, z16)
    h0 = _tc_mm(x, W0)
    dinv, g = _tc_scale0(degp, h0)
    for b, W in ((b0, W1), (b1, W2), (b2, W3)):
        p = _sc_agg(src2, dst2, g, z64)
        g = _tc_mid(p, g, dinv, b.reshape(1, D), W)
    p = _sc_agg(src2, dst2, g, z64)
    return _tc_last(p, g, dinv, b3.reshape(1, D))


# speedup vs baseline: 1.3948x; 1.3948x over previous
import functools

import jax
import jax.numpy as jnp
from jax.experimental import pallas as pl
from jax.experimental.pallas import tpu as pltpu
from jax.experimental.pallas import tpu_sc as plsc

N = 10000
E = 320000
D = 64
DG = 16
NC = 2
NS = 16
NT = NC * NS
CK = 128
CH = 80
EPT = CH * CK
EPAD = NT * EPT
VPAD = 10112
STRIPE = VPAD // NS
DUMMY_DST = 10008


def _mesh():
    return plsc.VectorSubcoreMesh(core_axis_name="c", subcore_axis_name="s")


_SC_PARAMS = pltpu.CompilerParams(use_tc_tiling_on_sc=False)


def _sc_degree(dst2, ones, z16):
    @functools.partial(
        pl.kernel,
        out_type=jax.ShapeDtypeStruct((NC, VPAD, DG), jnp.float32),
        mesh=_mesh(),
        scratch_types=[
            pltpu.VMEM((CH, CK), jnp.int32),
            pltpu.VMEM((CK, DG), jnp.float32),
            pltpu.VMEM_SHARED((VPAD, DG), jnp.float32),
            pltpu.SemaphoreType.DMA((4,)),
        ],
        compiler_params=_SC_PARAMS,
    )
    def k(dst_hbm, ones_hbm, z_hbm, out_hbm, didx_all, ones_v, acc, ssem):
        c = jax.lax.axis_index("c")
        s = jax.lax.axis_index("s")
        w = c * NS + s
        pltpu.sync_copy(ones_hbm, ones_v)
        pltpu.sync_copy(dst_hbm.at[pl.ds(w * CH, CH)], didx_all)
        pltpu.sync_copy(z_hbm.at[pl.ds(s * STRIPE, STRIPE)],
                        acc.at[pl.ds(s * STRIPE, STRIPE)])
        plsc.subcore_barrier()

        for b in range(4):
            pltpu.async_copy(ones_v, acc.at[didx_all.at[b]], ssem.at[b],
                             add=True)

        @pl.loop(0, CH // 4 - 1)
        def _(t):
            base = t * 4
            for b in range(4):
                pltpu.make_async_copy(ones_v, acc.at[didx_all.at[base + b]],
                                      ssem.at[b]).wait()
                pltpu.async_copy(ones_v, acc.at[didx_all.at[base + 4 + b]],
                                 ssem.at[b], add=True)

        for b in range(4):
            pltpu.make_async_copy(ones_v, acc.at[didx_all.at[b]],
                                  ssem.at[b]).wait()

        plsc.subcore_barrier()
        pltpu.sync_copy(acc.at[pl.ds(s * STRIPE, STRIPE)],
                        out_hbm.at[c].at[pl.ds(s * STRIPE, STRIPE)])

    return k(dst2, ones, z16)


NB = 2


def _sc_agg(src2, dst2, g, z64):
    @functools.partial(
        pl.kernel,
        out_type=jax.ShapeDtypeStruct((NC, VPAD, D), jnp.float32),
        mesh=_mesh(),
        scratch_types=[
            pltpu.VMEM((CH, CK), jnp.int32),
            pltpu.VMEM((CH, CK), jnp.int32),
            pltpu.VMEM((NB, CK, D), jnp.float32),
            pltpu.VMEM_SHARED((VPAD, D), jnp.float32),
            pltpu.VMEM_SHARED((VPAD, D), jnp.float32),
            pltpu.SemaphoreType.DMA((NB,)),
            pltpu.SemaphoreType.DMA((NB,)),
            pltpu.SemaphoreType.DMA((4,)),
        ],
        compiler_params=_SC_PARAMS,
    )
    def k(src_hbm, dst_hbm, g_hbm, z_hbm, out_hbm, sidx_all, didx_all,
          rows_v, g_sp, acc, gsem, ssem, psem):
        c = jax.lax.axis_index("c")
        s = jax.lax.axis_index("s")
        w = c * NS + s
        pltpu.async_copy(src_hbm.at[pl.ds(w * CH, CH)], sidx_all, psem.at[0])
        pltpu.async_copy(dst_hbm.at[pl.ds(w * CH, CH)], didx_all, psem.at[1])
        pltpu.async_copy(g_hbm.at[pl.ds(s * STRIPE, STRIPE)],
                         g_sp.at[pl.ds(s * STRIPE, STRIPE)], psem.at[2])
        pltpu.async_copy(z_hbm.at[pl.ds(s * STRIPE, STRIPE)],
                         acc.at[pl.ds(s * STRIPE, STRIPE)], psem.at[3])
        pltpu.make_async_copy(src_hbm.at[pl.ds(w * CH, CH)], sidx_all,
                              psem.at[0]).wait()
        pltpu.make_async_copy(dst_hbm.at[pl.ds(w * CH, CH)], didx_all,
                              psem.at[1]).wait()
        pltpu.make_async_copy(g_hbm.at[pl.ds(s * STRIPE, STRIPE)],
                              g_sp.at[pl.ds(s * STRIPE, STRIPE)],
                              psem.at[2]).wait()
        pltpu.make_async_copy(z_hbm.at[pl.ds(s * STRIPE, STRIPE)],
                              acc.at[pl.ds(s * STRIPE, STRIPE)],
                              psem.at[3]).wait()
        plsc.subcore_barrier()

        for b in range(NB):
            pltpu.async_copy(g_sp.at[sidx_all.at[b]], rows_v.at[b],
                             gsem.at[b])

        @pl.loop(0, CH // NB - 1)
        def _(t):
            base = t * NB
            for b in range(NB):
                pltpu.make_async_copy(g_sp.at[sidx_all.at[base + b]],
                                      rows_v.at[b], gsem.at[b]).wait()
                pltpu.async_copy(rows_v.at[b], acc.at[didx_all.at[base + b]],
                                 ssem.at[b], add=True)
            for b in range(NB):
                pltpu.make_async_copy(rows_v.at[b],
                                      acc.at[didx_all.at[base + b]],
                                      ssem.at[b]).wait()
                pltpu.async_copy(g_sp.at[sidx_all.at[base + NB + b]],
                                 rows_v.at[b], gsem.at[b])

        base = CH - NB
        for b in range(NB):
            pltpu.make_async_copy(g_sp.at[sidx_all.at[base + b]],
                                  rows_v.at[b], gsem.at[b]).wait()
            pltpu.async_copy(rows_v.at[b], acc.at[didx_all.at[base + b]],
                             ssem.at[b], add=True)
        for b in range(NB):
            pltpu.make_async_copy(rows_v.at[b], acc.at[didx_all.at[base + b]],
                                  ssem.at[b]).wait()

        plsc.subcore_barrier()
        pltpu.sync_copy(acc.at[pl.ds(s * STRIPE, STRIPE)],
                        out_hbm.at[c].at[pl.ds(s * STRIPE, STRIPE)])

    return k(src2, dst2, g, z64)


def _tc_mm(x, W):
    def body(x_ref, w_ref, o_ref):
        o_ref[...] = jnp.dot(x_ref[...], w_ref[...],
                             preferred_element_type=jnp.float32)

    return pl.pallas_call(
        body,
        out_shape=jax.ShapeDtypeStruct((x.shape[0], W.shape[1]), jnp.float32),
    )(x, W)


def _tc_scale0(degp, h0):
    def body(dp_ref, h_ref, dinv_ref, g_ref):
        dp = dp_ref[...]
        deg = dp[0, :N, 0:1] + dp[1, :N, 0:1] + 1.0
        dinv = jax.lax.rsqrt(deg)
        dinv_ref[...] = dinv
        g_ref[...] = jnp.concatenate(
            [h_ref[...] * dinv, jnp.zeros((VPAD - N, D), jnp.float32)])

    return pl.pallas_call(
        body,
        out_shape=[
            jax.ShapeDtypeStruct((N, 1), jnp.float32),
            jax.ShapeDtypeStruct((VPAD, D), jnp.float32),
        ],
    )(degp, h0)


def _tc_mid(p, g, dinv, b, W):
    def body(p_ref, g_ref, dinv_ref, b_ref, w_ref, o_ref):
        pv = p_ref[...]
        sm = pv[0, :N] + pv[1, :N] + g_ref[...][:N]
        dv = dinv_ref[...]
        f = jnp.maximum(dv * sm + b_ref[...], 0.0)
        o_ref[...] = jnp.concatenate(
            [dv * jnp.dot(f, w_ref[...], preferred_element_type=jnp.float32),
             jnp.zeros((VPAD - N, D), jnp.float32)])

    return pl.pallas_call(
        body,
        out_shape=jax.ShapeDtypeStruct((VPAD, D), jnp.float32),
    )(p, g, dinv, b, W)


def _tc_last(p, g, dinv, b):
    def body(p_ref, g_ref, dinv_ref, b_ref, o_ref):
        pv = p_ref[...]
        sm = pv[0, :N] + pv[1, :N] + g_ref[...][:N]
        o_ref[...] = dinv_ref[...] * sm + b_ref[...]

    return pl.pallas_call(
        body,
        out_shape=jax.ShapeDtypeStruct((N, D), jnp.float32),
    )(p, g, dinv, b)


def kernel(x, edge_index, W0, b0, W1, b1, W2, b2, W3, b3):
    src = edge_index[0].astype(jnp.int32)
    dst = edge_index[1].astype(jnp.int32)
    pad = EPAD - E
    src2 = jnp.concatenate([src, jnp.zeros((pad,), jnp.int32)]).reshape(
        NT * CH, CK)
    dst2 = jnp.concatenate([dst, jnp.full((pad,), DUMMY_DST, jnp.int32)]
                           ).reshape(NT * CH, CK)
    ones = jnp.ones((CK, DG), jnp.float32)
    z16 = jnp.zeros((VPAD, DG), jnp.float32)
    z64 = jnp.zeros((VPAD, D), jnp.float32)

    degp = _sc_degree(dst2, ones, z16)
    h0 = _tc_mm(x, W0)
    dinv, g = _tc_scale0(degp, h0)
    for b, W in ((b0, W1), (b1, W2), (b2, W3)):
        p = _sc_agg(src2, dst2, g, z64)
        g = _tc_mid(p, g, dinv, b.reshape(1, D), W)
    p = _sc_agg(src2, dst2, g, z64)
    return _tc_last(p, g, dinv, b3.reshape(1, D))


# speedup vs baseline: 1.3978x; 1.0022x over previous
import functools

import jax
import jax.numpy as jnp
from jax.experimental import pallas as pl
from jax.experimental.pallas import tpu as pltpu
from jax.experimental.pallas import tpu_sc as plsc

N = 10000
E = 320000
D = 64
DG = 16
NC = 2
NS = 16
NT = NC * NS
CK = 128
CH = 80
EPT = CH * CK
EPAD = NT * EPT
VPAD = 10112
STRIPE = VPAD // NS
DUMMY_DST = 10008


def _mesh():
    return plsc.VectorSubcoreMesh(core_axis_name="c", subcore_axis_name="s")


_SC_PARAMS = pltpu.CompilerParams(use_tc_tiling_on_sc=False)


def _sc_degree(dst2, ones, z16):
    @functools.partial(
        pl.kernel,
        out_type=jax.ShapeDtypeStruct((NC, VPAD, DG), jnp.float32),
        mesh=_mesh(),
        scratch_types=[
            pltpu.VMEM((CH, CK), jnp.int32),
            pltpu.VMEM((CK, DG), jnp.float32),
            pltpu.VMEM_SHARED((VPAD, DG), jnp.float32),
            pltpu.SemaphoreType.DMA((4,)),
        ],
        compiler_params=_SC_PARAMS,
    )
    def k(dst_hbm, ones_hbm, z_hbm, out_hbm, didx_all, ones_v, acc, ssem):
        c = jax.lax.axis_index("c")
        s = jax.lax.axis_index("s")
        w = c * NS + s
        pltpu.sync_copy(ones_hbm, ones_v)
        pltpu.sync_copy(dst_hbm.at[pl.ds(w * CH, CH)], didx_all)
        pltpu.sync_copy(z_hbm.at[pl.ds(s * STRIPE, STRIPE)],
                        acc.at[pl.ds(s * STRIPE, STRIPE)])
        plsc.subcore_barrier()

        for b in range(4):
            pltpu.async_copy(ones_v, acc.at[didx_all.at[b]], ssem.at[b],
                             add=True)

        @pl.loop(0, CH // 4 - 1)
        def _(t):
            base = t * 4
            for b in range(4):
                pltpu.make_async_copy(ones_v, acc.at[didx_all.at[base + b]],
                                      ssem.at[b]).wait()
                pltpu.async_copy(ones_v, acc.at[didx_all.at[base + 4 + b]],
                                 ssem.at[b], add=True)

        for b in range(4):
            pltpu.make_async_copy(ones_v, acc.at[didx_all.at[b]],
                                  ssem.at[b]).wait()

        plsc.subcore_barrier()
        pltpu.sync_copy(acc.at[pl.ds(s * STRIPE, STRIPE)],
                        out_hbm.at[c].at[pl.ds(s * STRIPE, STRIPE)])

    return k(dst2, ones, z16)


NB = 2


def _sc_agg(src2, dst2, g, z64):
    @functools.partial(
        pl.kernel,
        out_type=jax.ShapeDtypeStruct((NC, VPAD, D), jnp.float32),
        mesh=_mesh(),
        scratch_types=[
            pltpu.VMEM((CH, CK), jnp.int32),
            pltpu.VMEM((CH, CK), jnp.int32),
            pltpu.VMEM((NB, CK, D), jnp.float32),
            pltpu.VMEM_SHARED((VPAD, D), jnp.float32),
            pltpu.VMEM_SHARED((VPAD, D), jnp.float32),
            pltpu.SemaphoreType.DMA((NB,)),
            pltpu.SemaphoreType.DMA((NB,)),
            pltpu.SemaphoreType.DMA((4,)),
        ],
        compiler_params=_SC_PARAMS,
    )
    def k(src_hbm, dst_hbm, g_hbm, z_hbm, out_hbm, sidx_all, didx_all,
          rows_v, g_sp, acc, gsem, ssem, psem):
        c = jax.lax.axis_index("c")
        s = jax.lax.axis_index("s")
        w = c * NS + s
        pltpu.async_copy(src_hbm.at[pl.ds(w * CH, CH)], sidx_all, psem.at[0])
        pltpu.async_copy(dst_hbm.at[pl.ds(w * CH, CH)], didx_all, psem.at[1])
        pltpu.async_copy(g_hbm.at[pl.ds(s * STRIPE, STRIPE)],
                         g_sp.at[pl.ds(s * STRIPE, STRIPE)], psem.at[2])
        pltpu.async_copy(z_hbm.at[pl.ds(s * STRIPE, STRIPE)],
                         acc.at[pl.ds(s * STRIPE, STRIPE)], psem.at[3])
        pltpu.make_async_copy(src_hbm.at[pl.ds(w * CH, CH)], sidx_all,
                              psem.at[0]).wait()
        pltpu.make_async_copy(dst_hbm.at[pl.ds(w * CH, CH)], didx_all,
                              psem.at[1]).wait()
        pltpu.make_async_copy(g_hbm.at[pl.ds(s * STRIPE, STRIPE)],
                              g_sp.at[pl.ds(s * STRIPE, STRIPE)],
                              psem.at[2]).wait()
        pltpu.make_async_copy(z_hbm.at[pl.ds(s * STRIPE, STRIPE)],
                              acc.at[pl.ds(s * STRIPE, STRIPE)],
                              psem.at[3]).wait()
        plsc.subcore_barrier()

        for b in range(NB):
            pltpu.async_copy(g_sp.at[sidx_all.at[b]], rows_v.at[b],
                             gsem.at[b])

        @pl.loop(0, CH // NB - 1)
        def _(t):
            base = t * NB
            for b in range(NB):
                pltpu.make_async_copy(g_sp.at[sidx_all.at[base + b]],
                                      rows_v.at[b], gsem.at[b]).wait()
                pltpu.async_copy(rows_v.at[b], acc.at[didx_all.at[base + b]],
                                 ssem.at[b], add=True)
            for b in range(NB):
                pltpu.make_async_copy(rows_v.at[b],
                                      acc.at[didx_all.at[base + b]],
                                      ssem.at[b]).wait()
                pltpu.async_copy(g_sp.at[sidx_all.at[base + NB + b]],
                                 rows_v.at[b], gsem.at[b])

        base = CH - NB
        for b in range(NB):
            pltpu.make_async_copy(g_sp.at[sidx_all.at[base + b]],
                                  rows_v.at[b], gsem.at[b]).wait()
            pltpu.async_copy(rows_v.at[b], acc.at[didx_all.at[base + b]],
                             ssem.at[b], add=True)
        for b in range(NB):
            pltpu.make_async_copy(rows_v.at[b], acc.at[didx_all.at[base + b]],
                                  ssem.at[b]).wait()

        plsc.subcore_barrier()
        pltpu.sync_copy(acc.at[pl.ds(s * STRIPE, STRIPE)],
                        out_hbm.at[c].at[pl.ds(s * STRIPE, STRIPE)])

    return k(src2, dst2, g, z64)


def _tc_scale0(degp, x, W0):
    def body(dp_ref, x_ref, w_ref, dinv_ref, g_ref):
        dp = dp_ref[...]
        deg = dp[0, :N, 0:1] + dp[1, :N, 0:1] + 1.0
        dinv = jax.lax.rsqrt(deg)
        dinv_ref[...] = dinv
        h = jnp.dot(x_ref[...], w_ref[...],
                    preferred_element_type=jnp.float32)
        g_ref[...] = jnp.concatenate(
            [h * dinv, jnp.zeros((VPAD - N, D), jnp.float32)])

    return pl.pallas_call(
        body,
        out_shape=[
            jax.ShapeDtypeStruct((N, 1), jnp.float32),
            jax.ShapeDtypeStruct((VPAD, D), jnp.float32),
        ],
    )(degp, x, W0)


def _tc_mid(p, g, dinv, b, W):
    def body(p_ref, g_ref, dinv_ref, b_ref, w_ref, o_ref):
        pv = p_ref[...]
        sm = pv[0, :N] + pv[1, :N] + g_ref[...][:N]
        dv = dinv_ref[...]
        f = jnp.maximum(dv * sm + b_ref[...], 0.0)
        o_ref[...] = jnp.concatenate(
            [dv * jnp.dot(f, w_ref[...], preferred_element_type=jnp.float32),
             jnp.zeros((VPAD - N, D), jnp.float32)])

    return pl.pallas_call(
        body,
        out_shape=jax.ShapeDtypeStruct((VPAD, D), jnp.float32),
    )(p, g, dinv, b, W)


def _tc_last(p, g, dinv, b):
    def body(p_ref, g_ref, dinv_ref, b_ref, o_ref):
        pv = p_ref[...]
        sm = pv[0, :N] + pv[1, :N] + g_ref[...][:N]
        o_ref[...] = dinv_ref[...] * sm + b_ref[...]

    return pl.pallas_call(
        body,
        out_shape=jax.ShapeDtypeStruct((N, D), jnp.float32),
    )(p, g, dinv, b)


def kernel(x, edge_index, W0, b0, W1, b1, W2, b2, W3, b3):
    src = edge_index[0].astype(jnp.int32)
    dst = edge_index[1].astype(jnp.int32)
    pad = EPAD - E
    src2 = jnp.concatenate([src, jnp.zeros((pad,), jnp.int32)]).reshape(
        NT * CH, CK)
    dst2 = jnp.concatenate([dst, jnp.full((pad,), DUMMY_DST, jnp.int32)]
                           ).reshape(NT * CH, CK)
    ones = jnp.ones((CK, DG), jnp.float32)
    z16 = jnp.zeros((VPAD, DG), jnp.float32)
    z64 = jnp.zeros((VPAD, D), jnp.float32)

    degp = _sc_degree(dst2, ones, z16)
    dinv, g = _tc_scale0(degp, x, W0)
    for b, W in ((b0, W1), (b1, W2), (b2, W3)):
        p = _sc_agg(src2, dst2, g, z64)
        g = _tc_mid(p, g, dinv, b.reshape(1, D), W)
    p = _sc_agg(src2, dst2, g, z64)
    return _tc_last(p, g, dinv, b3.reshape(1, D))
